# Initial kernel scaffold; baseline (speedup 1.0000x reference)
import jax
import jax.numpy as jnp
from jax.experimental import pallas as pl


def kernel(x, ln_g, ln_b, Wr, br, W1, b1, W2, b2):
    raise NotImplementedError("write your pallas kernel here")


# speedup vs baseline: 22.6964x; 22.6964x over previous
import jax
import jax.numpy as jnp
from jax.experimental import pallas as pl

_NE = 8
_DM = 768
_DF = 3072
_SCALE = 0.8
_SPARSITY = 0.1
_T = 4096
_TB = 256

_ABS_MASK = 0x7FFFFFFF


def _select2(w_ref, r_bot, r_top):
    def bits():
        return jax.lax.bitcast_convert_type(w_ref[...], jnp.int32) & _ABS_MASK

    def body(_, carry):
        lo_b, hi_b, lo_t, hi_t = carry
        mid_b = (lo_b + hi_b) >> 1
        mid_t = (lo_t + hi_t) >> 1
        b = bits()
        c_b = jnp.sum((b <= mid_b).astype(jnp.int32))
        c_t = jnp.sum((b <= mid_t).astype(jnp.int32))
        ge_b = c_b >= r_bot
        ge_t = c_t >= r_top
        return (
            jnp.where(ge_b, lo_b, mid_b),
            jnp.where(ge_b, mid_b, hi_b),
            jnp.where(ge_t, lo_t, mid_t),
            jnp.where(ge_t, mid_t, hi_t),
        )

    init = (jnp.int32(-1), jnp.int32(0x7FFFFFFF), jnp.int32(-1), jnp.int32(0x7FFFFFFF))
    _, hi_b, _, hi_t = jax.lax.fori_loop(0, 31, body, init)
    return hi_b, hi_t


def _apply_pert(w_ref, hi_b, hi_t, out_dtype):
    b = jax.lax.bitcast_convert_type(w_ref[...], jnp.int32) & _ABS_MASK
    bot = (b <= hi_b).astype(jnp.float32)
    top = (b >= hi_t).astype(jnp.float32)
    return (w_ref[...] * (1.0 + _SCALE * (bot - top))).astype(out_dtype)


def _pert_body(w1_ref, b1_ref, w2_ref, b2_ref, pw1_ref, pb1_ref, pw2_ref, pb2_ref):
    for w_ref, p_ref, n, dt in (
        (w1_ref, pw1_ref, _DM * _DF, jnp.bfloat16),
        (b1_ref, pb1_ref, _DF, jnp.float32),
        (w2_ref, pw2_ref, _DF * _DM, jnp.bfloat16),
        (b2_ref, pb2_ref, _DM, jnp.float32),
    ):
        k = max(1, int(n * _SPARSITY / 2))
        hi_b, hi_t = _select2(w_ref, jnp.int32(k), jnp.int32(n - k + 1))
        p_ref[...] = _apply_pert(w_ref, hi_b, hi_t, dt)


def _moe_body(
    x_ref, g_ref, bt_ref, wr_ref, br_ref,
    w1_ref, b1_ref, w2_ref, b2_ref,
    pw1_ref, pb1_ref, pw2_ref, pb2_ref,
    o_ref,
):
    xb = x_ref[...]
    m = jnp.mean(xb, axis=-1, keepdims=True)
    xc = xb - m
    v = jnp.mean(xc * xc, axis=-1, keepdims=True)
    h = xc * jax.lax.rsqrt(v + 1e-5) * g_ref[...] + bt_ref[...]
    logits = (
        jnp.dot(h.astype(jnp.bfloat16), wr_ref[...].astype(jnp.bfloat16),
                preferred_element_type=jnp.float32)
        + br_ref[...]
    )
    mx = jnp.max(logits, axis=-1, keepdims=True)
    e = jnp.exp(logits - mx)
    p0 = e[:, :1] / jnp.sum(e, axis=-1, keepdims=True)
    pr = 1.0 - p0
    xb16 = xb.astype(jnp.bfloat16)
    hc = jax.nn.gelu(
        jnp.dot(xb16, w1_ref[...], preferred_element_type=jnp.float32) + b1_ref[...]
    )
    hp = jax.nn.gelu(
        jnp.dot(xb16, pw1_ref[...], preferred_element_type=jnp.float32) + pb1_ref[...]
    )
    acc = (
        jnp.dot((p0 * hc).astype(jnp.bfloat16), w2_ref[...],
                preferred_element_type=jnp.float32)
        + jnp.dot((pr * hp).astype(jnp.bfloat16), pw2_ref[...],
                  preferred_element_type=jnp.float32)
    )
    o_ref[...] = acc + p0 * b2_ref[...] + pr * pb2_ref[...]


def kernel(x, ln_g, ln_b, Wr, br, W1, b1, W2, b2):
    b1r = b1.reshape(1, _DF)
    b2r = b2.reshape(1, _DM)
    pW1, pb1, pW2, pb2 = pl.pallas_call(
        _pert_body,
        out_shape=[
            jax.ShapeDtypeStruct((_DM, _DF), jnp.bfloat16),
            jax.ShapeDtypeStruct((1, _DF), jnp.float32),
            jax.ShapeDtypeStruct((_DF, _DM), jnp.bfloat16),
            jax.ShapeDtypeStruct((1, _DM), jnp.float32),
        ],
    )(W1, b1r, W2, b2r)

    full = lambda shape: pl.BlockSpec(shape, lambda i: (0, 0))
    out = pl.pallas_call(
        _moe_body,
        grid=(_T // _TB,),
        in_specs=[
            pl.BlockSpec((_TB, _DM), lambda i: (i, 0)),
            full((1, _DM)),
            full((1, _DM)),
            full((_DM, _NE)),
            full((1, _NE)),
            full((_DM, _DF)),
            full((1, _DF)),
            full((_DF, _DM)),
            full((1, _DM)),
            full((_DM, _DF)),
            full((1, _DF)),
            full((_DF, _DM)),
            full((1, _DM)),
        ],
        out_specs=pl.BlockSpec((_TB, _DM), lambda i: (i, 0)),
        out_shape=jax.ShapeDtypeStruct((_T, _DM), jnp.float32),
    )(
        x, ln_g.reshape(1, _DM), ln_b.reshape(1, _DM), Wr, br.reshape(1, _NE),
        W1.astype(jnp.bfloat16), b1r, W2.astype(jnp.bfloat16), b2r,
        pW1, pb1, pW2, pb2,
    )
    return out
